# Initial kernel scaffold; baseline (speedup 1.0000x reference)
import jax
import jax.numpy as jnp
from jax.experimental import pallas as pl


def kernel(entity, adj_entity, adj_relation):
    raise NotImplementedError("write your pallas kernel here")


# speedup vs baseline: 8.7671x; 8.7671x over previous
import jax
import jax.numpy as jnp
from jax import lax
from jax.experimental import pallas as pl
from jax.experimental.pallas import tpu as pltpu
from jax.experimental.pallas import tpu_sc as plsc

_NB = 32
_BATCH = 16384
_NC = 2
_NS = 16
_NW = _NC * _NS
_G = 128


def _mesh():
  return plsc.VectorSubcoreMesh(
      core_axis_name="c", subcore_axis_name="s",
      num_cores=_NC, num_subcores=_NS)


def _wid():
  return lax.axis_index("s") * _NC + lax.axis_index("c")


def _gather_body(n_per_w, idx_hbm, adj_e_hbm, adj_r_hbm,
                 e_out, r_out, idx_v, ebuf, rbuf, sem_e, sem_r):
  base = _wid() * n_per_w
  pltpu.sync_copy(idx_hbm.at[pl.ds(base, n_per_w)], idx_v)

  def group_body(g, carry):
    off = idx_v.at[pl.ds(g * _G, _G)]
    ce = pltpu.async_copy(adj_e_hbm.at[off], ebuf, sem_e)
    cr = pltpu.async_copy(adj_r_hbm.at[off], rbuf, sem_r)
    out0 = base + g * _G
    ce.wait()
    pltpu.sync_copy(ebuf, e_out.at[pl.ds(out0, _G)])
    cr.wait()
    pltpu.sync_copy(rbuf, r_out.at[pl.ds(out0, _G)])
    return carry

  lax.fori_loop(0, n_per_w // _G, group_body, 0)


def _hop(idx_flat, adj_entity, adj_relation):
  n = idx_flat.shape[0]
  n_per_w = n // _NW
  import functools
  body = functools.partial(_gather_body, n_per_w)
  out_type = (
      jax.ShapeDtypeStruct((n, _NB), jnp.int32),
      jax.ShapeDtypeStruct((n, _NB), jnp.int32),
  )
  scratch = [
      pltpu.VMEM((n_per_w,), jnp.int32),
      pltpu.VMEM((_G, _NB), jnp.int32),
      pltpu.VMEM((_G, _NB), jnp.int32),
      pltpu.SemaphoreType.DMA,
      pltpu.SemaphoreType.DMA,
  ]
  return pl.kernel(
      body, out_type=out_type, mesh=_mesh(), scratch_types=scratch,
      compiler_params=pltpu.CompilerParams(use_tc_tiling_on_sc=False),
  )(idx_flat, adj_entity, adj_relation)


def kernel(entity, adj_entity, adj_relation):
  ent1, rel1 = _hop(entity.reshape(-1), adj_entity, adj_relation)
  ent2, rel2 = _hop(ent1.reshape(-1), adj_entity, adj_relation)
  return (entity,
          ent1,
          ent2.reshape(_BATCH, _NB * _NB),
          rel1,
          rel2.reshape(_BATCH, _NB * _NB))


# speedup vs baseline: 10.8809x; 1.2411x over previous
import functools

import jax
import jax.numpy as jnp
from jax import lax
from jax.experimental import pallas as pl
from jax.experimental.pallas import tpu as pltpu
from jax.experimental.pallas import tpu_sc as plsc

_NB = 32
_BATCH = 16384
_NC = 2
_NS = 16
_NW = _NC * _NS
_G = 128
_L = 4
_S = 2 * _L


def _mesh():
  return plsc.VectorSubcoreMesh(
      core_axis_name="c", subcore_axis_name="s",
      num_cores=_NC, num_subcores=_NS)


def _gather_body(n_per_w, idx_hbm, adj_e_hbm, adj_r_hbm,
                 e_out, r_out, idx_v, ebuf, rbuf, *sems):
  gsems, wsems = sems[:_S], sems[_S:]
  wid = lax.axis_index("s") * _NC + lax.axis_index("c")
  base = wid * n_per_w
  pltpu.sync_copy(idx_hbm.at[pl.ds(base, n_per_w)], idx_v)
  ng = n_per_w // _G

  def fire_gathers(g, slot):
    off = idx_v.at[pl.ds(g * _G, _G)]
    pltpu.async_copy(adj_e_hbm.at[off], ebuf.at[slot], gsems[slot])
    pltpu.async_copy(adj_r_hbm.at[off], rbuf.at[slot], gsems[slot])

  def wait_gathers(g, slot):
    out0 = base + g * _G
    pltpu.make_async_copy(
        e_out.at[pl.ds(out0, _G)], ebuf.at[slot], gsems[slot]).wait()
    pltpu.make_async_copy(
        r_out.at[pl.ds(out0, _G)], rbuf.at[slot], gsems[slot]).wait()

  def fire_writes(g, slot):
    out0 = base + g * _G
    pltpu.async_copy(ebuf.at[slot], e_out.at[pl.ds(out0, _G)], wsems[slot])
    pltpu.async_copy(rbuf.at[slot], r_out.at[pl.ds(out0, _G)], wsems[slot])

  def wait_writes(g, slot):
    out0 = base + g * _G
    pltpu.make_async_copy(
        ebuf.at[slot], e_out.at[pl.ds(out0, _G)], wsems[slot]).wait()
    pltpu.make_async_copy(
        rbuf.at[slot], r_out.at[pl.ds(out0, _G)], wsems[slot]).wait()

  if ng <= _S:
    for g in range(ng):
      fire_gathers(g, g)
    for g in range(ng):
      wait_gathers(g, g)
      fire_writes(g, g)
    for g in range(ng):
      wait_writes(g, g)
    return

  for b in range(_L):
    fire_gathers(b, b)
  for b in range(_S):
    g = b
    slot_n = (b + _L) % _S
    if g + _L >= _S:
      wait_writes(g - _L, slot_n)
    fire_gathers(g + _L, slot_n)
    wait_gathers(g, b)
    fire_writes(g, b)

  def outer(t, carry):
    for b in range(_S):
      g = t * _S + b
      slot_n = (b + _L) % _S
      wait_writes(g - _L, slot_n)
      fire_gathers(g + _L, slot_n)
      wait_gathers(g, b)
      fire_writes(g, b)
    return carry

  lax.fori_loop(1, ng // _S - 1, outer, 0)

  t_last = ng // _S - 1
  for b in range(_S):
    g = t_last * _S + b
    slot_n = (b + _L) % _S
    if g + _L < ng:
      wait_writes(g - _L, slot_n)
      fire_gathers(g + _L, slot_n)
    wait_gathers(g, b)
    fire_writes(g, b)
  for b in range(_S):
    wait_writes(ng - _S + b, b)


def _hop(idx_flat, adj_entity, adj_relation):
  n = idx_flat.shape[0]
  n_per_w = n // _NW
  body = functools.partial(_gather_body, n_per_w)
  out_type = (
      jax.ShapeDtypeStruct((n, _NB), jnp.int32),
      jax.ShapeDtypeStruct((n, _NB), jnp.int32),
  )
  scratch = [
      pltpu.VMEM((n_per_w,), jnp.int32),
      pltpu.VMEM((_S, _G, _NB), jnp.int32),
      pltpu.VMEM((_S, _G, _NB), jnp.int32),
  ] + [pltpu.SemaphoreType.DMA] * (2 * _S)
  return pl.kernel(
      body, out_type=out_type, mesh=_mesh(), scratch_types=scratch,
      compiler_params=pltpu.CompilerParams(use_tc_tiling_on_sc=False),
  )(idx_flat, adj_entity, adj_relation)


def kernel(entity, adj_entity, adj_relation):
  ent1, rel1 = _hop(entity.reshape(-1), adj_entity, adj_relation)
  ent2, rel2 = _hop(ent1.reshape(-1), adj_entity, adj_relation)
  return (entity,
          ent1,
          ent2.reshape(_BATCH, _NB * _NB),
          rel1,
          rel2.reshape(_BATCH, _NB * _NB))
